# baseline (speedup 1.0000x reference)
import functools

import jax
import jax.numpy as jnp
from jax import lax
from jax.experimental import pallas as pl
from jax.experimental.pallas import tpu as pltpu
from jax.experimental.pallas import tpu_sc as plsc

_B, _S, _D = 16, 576, 256
_N = _B * _S
_M = 8192
_TT = 512
_TM = 1024
_NT = _N // _TT
_NM = _M // _TM


def _argmin_body(res_ref, wt_ref, r2_ref, w2_ref, idx_ref, bd_ref, bi_ref):
    mj = pl.program_id(1)

    @pl.when(mj == 0)
    def _init():
        bd_ref[...] = jnp.full_like(bd_ref[...], jnp.inf)
        bi_ref[...] = jnp.zeros_like(bi_ref[...])

    cross = lax.dot_general(
        res_ref[...], wt_ref[...], (((1,), (0,)), ((), ())),
        preferred_element_type=jnp.float32)
    d2 = r2_ref[...] - 2.0 * cross + w2_ref[...]
    dmin = jnp.min(d2, axis=1, keepdims=True)
    ii = lax.broadcasted_iota(jnp.int32, d2.shape, 1)
    li = jnp.min(jnp.where(d2 == dmin, ii, _TM), axis=1, keepdims=True)
    gi = li + mj * _TM
    take = dmin < bd_ref[...]
    bi_ref[...] = jnp.where(take, gi, bi_ref[...])
    bd_ref[...] = jnp.where(take, dmin, bd_ref[...])

    @pl.when(mj == pl.num_programs(1) - 1)
    def _emit():
        idx_ref[...] = bi_ref[...].reshape(1, _TT, 1)


def _argmin_call(res, wt, r2, w2):
    out = pl.pallas_call(
        _argmin_body,
        grid=(_NT, _NM),
        in_specs=[
            pl.BlockSpec((_TT, _D), lambda i, j: (i, 0)),
            pl.BlockSpec((_D, _TM), lambda i, j: (0, j)),
            pl.BlockSpec((_TT, 1), lambda i, j: (i, 0)),
            pl.BlockSpec((1, _TM), lambda i, j: (0, j)),
        ],
        out_specs=pl.BlockSpec((1, _TT, 1), lambda i, j: (i, 0, 0)),
        out_shape=jax.ShapeDtypeStruct((_NT, _TT, 1), jnp.int32),
        scratch_shapes=[
            pltpu.VMEM((_TT, 1), jnp.float32),
            pltpu.VMEM((_TT, 1), jnp.int32),
        ],
        compiler_params=pltpu.CompilerParams(
            dimension_semantics=("parallel", "arbitrary")),
    )(res, wt, r2, w2)
    return out.reshape(_N)


_NC, _NS, _L = 2, 16, 16
_NW = _NC * _NS
_PW = _N // _NW
_CH = 96
_NCH = _PW // _CH

_sc_mesh = plsc.VectorSubcoreMesh(core_axis_name="c", subcore_axis_name="s")


def _scsub_body(w_hbm, idx_hbm, res_hbm, out_hbm, idxv, rowsv, resv, sem):
    wid = lax.axis_index("s") * _NC + lax.axis_index("c")
    base = wid * _PW
    for ch in range(_NCH):
        off = base + ch * _CH
        pltpu.sync_copy(idx_hbm.at[pl.ds(off, _CH)], idxv)
        gather = pltpu.async_copy(w_hbm.at[idxv], rowsv, sem)
        pltpu.sync_copy(res_hbm.at[pl.ds(off, _CH)], resv)
        gather.wait()

        def _row(r, carry):
            for j in range(_D // _L):
                sl = pl.ds(j * _L, _L)
                resv[r, sl] = resv[r, sl] - rowsv[r, sl]
            return carry

        lax.fori_loop(0, _CH, _row, 0)
        pltpu.sync_copy(resv, out_hbm.at[pl.ds(off, _CH)])


def _sc_residual_update(codebook, idx, res):
    fn = pl.kernel(
        _scsub_body,
        out_type=jax.ShapeDtypeStruct((_N, _D), jnp.float32),
        mesh=_sc_mesh,
        scratch_types=[
            pltpu.VMEM((_CH,), jnp.int32),
            pltpu.VMEM((_CH, _D), jnp.float32),
            pltpu.VMEM((_CH, _D), jnp.float32),
            pltpu.SemaphoreType.DMA,
        ],
    )
    return fn(codebook, idx, res)


def kernel(x, codebooks):
    x2 = x.reshape(_N, _D)
    residual = x2
    indices = []
    for k in range(codebooks.shape[0]):
        w = codebooks[k]
        r2 = jnp.sum(residual * residual, axis=-1, keepdims=True)
        w2 = jnp.sum(w * w, axis=-1).reshape(1, _M)
        idx = _argmin_call(residual, w.T, r2, w2)
        indices.append(idx.reshape(_B, _S))
        residual = _sc_residual_update(w, idx, residual)
    quantized = (x2 - residual).reshape(_B, _S, _D)
    return (quantized, jnp.stack(indices, axis=0))


# speedup vs baseline: 1.0490x; 1.0490x over previous
import functools

import jax
import jax.numpy as jnp
from jax import lax
from jax.experimental import pallas as pl
from jax.experimental.pallas import tpu as pltpu
from jax.experimental.pallas import tpu_sc as plsc

_B, _S, _D = 16, 576, 256
_N = _B * _S
_M = 8192
_TT = 512
_TM = 1024
_NT = _N // _TT
_NM = _M // _TM


def _argmin_body(res_ref, wt2_ref, r2_ref, w2_ref, idx_ref, bd_ref, bi_ref):
    mj = pl.program_id(1)

    cross2 = lax.dot_general(
        res_ref[...], wt2_ref[...], (((1,), (0,)), ((), ())),
        preferred_element_type=jnp.float32)
    d2 = r2_ref[...] - cross2 + w2_ref[...]
    take = jnp.logical_or(mj == 0, d2 < bd_ref[...])
    bi_ref[...] = jnp.where(take, jnp.int32(mj * _TM), bi_ref[...])
    bd_ref[...] = jnp.where(take, d2, bd_ref[...])

    @pl.when(mj == pl.num_programs(1) - 1)
    def _emit():
        bd = bd_ref[...]
        dmin = jnp.min(bd, axis=1, keepdims=True)
        jj = lax.broadcasted_iota(jnp.int32, bd.shape, 1)
        gi = bi_ref[...] + jj
        ei = jnp.where(bd == dmin, gi, jnp.int32(_M))
        idx_ref[...] = jnp.min(ei, axis=1, keepdims=True).reshape(1, _TT, 1)


def _argmin_call(res, wt, r2, w2):
    out = pl.pallas_call(
        _argmin_body,
        grid=(_NT, _NM),
        in_specs=[
            pl.BlockSpec((_TT, _D), lambda i, j: (i, 0)),
            pl.BlockSpec((_D, _TM), lambda i, j: (0, j)),
            pl.BlockSpec((_TT, 1), lambda i, j: (i, 0)),
            pl.BlockSpec((1, _TM), lambda i, j: (0, j)),
        ],
        out_specs=pl.BlockSpec((1, _TT, 1), lambda i, j: (i, 0, 0)),
        out_shape=jax.ShapeDtypeStruct((_NT, _TT, 1), jnp.int32),
        scratch_shapes=[
            pltpu.VMEM((_TT, _TM), jnp.float32),
            pltpu.VMEM((_TT, _TM), jnp.int32),
        ],
        compiler_params=pltpu.CompilerParams(
            dimension_semantics=("parallel", "arbitrary")),
    )(res, wt, r2, w2)
    return out.reshape(_N)


_NC, _NS, _L = 2, 16, 16
_NW = _NC * _NS
_PW = _N // _NW
_CH = 96
_NCH = _PW // _CH

_sc_mesh = plsc.VectorSubcoreMesh(core_axis_name="c", subcore_axis_name="s")


def _scsub_body(w_hbm, idx_hbm, res_hbm, out_hbm, idxv, rowsv, resv, sem):
    wid = lax.axis_index("s") * _NC + lax.axis_index("c")
    base = wid * _PW
    for ch in range(_NCH):
        off = base + ch * _CH
        pltpu.sync_copy(idx_hbm.at[pl.ds(off, _CH)], idxv)
        gather = pltpu.async_copy(w_hbm.at[idxv], rowsv, sem)
        pltpu.sync_copy(res_hbm.at[pl.ds(off, _CH)], resv)
        gather.wait()

        def _row(r, carry):
            for j in range(_D // _L):
                sl = pl.ds(j * _L, _L)
                resv[r, sl] = resv[r, sl] - rowsv[r, sl]
            return carry

        lax.fori_loop(0, _CH, _row, 0)
        pltpu.sync_copy(resv, out_hbm.at[pl.ds(off, _CH)])


def _sc_residual_update(codebook, idx, res):
    fn = pl.kernel(
        _scsub_body,
        out_type=jax.ShapeDtypeStruct((_N, _D), jnp.float32),
        mesh=_sc_mesh,
        scratch_types=[
            pltpu.VMEM((_CH,), jnp.int32),
            pltpu.VMEM((_CH, _D), jnp.float32),
            pltpu.VMEM((_CH, _D), jnp.float32),
            pltpu.SemaphoreType.DMA,
        ],
    )
    return fn(codebook, idx, res)


def kernel(x, codebooks):
    x2 = x.reshape(_N, _D)
    residual = x2
    indices = []
    for k in range(codebooks.shape[0]):
        w = codebooks[k]
        r2 = jnp.sum(residual * residual, axis=-1, keepdims=True)
        w2 = jnp.sum(w * w, axis=-1).reshape(1, _M)
        idx = _argmin_call(residual, (2.0 * w).T, r2, w2)
        indices.append(idx.reshape(_B, _S))
        residual = _sc_residual_update(w, idx, residual)
    quantized = (x2 - residual).reshape(_B, _S, _D)
    return (quantized, jnp.stack(indices, axis=0))


# speedup vs baseline: 1.3529x; 1.2897x over previous
import functools

import jax
import jax.numpy as jnp
from jax import lax
from jax.experimental import pallas as pl
from jax.experimental.pallas import tpu as pltpu
from jax.experimental.pallas import tpu_sc as plsc

_B, _S, _D = 16, 576, 256
_N = _B * _S
_M = 8192
_TM = 256
_NM = _M // _TM
_NCHAIN = 2


def _argmin_body(res_ref, wt2_ref, r2_ref, w2_ref, idx_ref, bd_ref, bi_ref):
    mj = pl.program_id(0)

    cross2 = lax.dot_general(
        res_ref[...], wt2_ref[...], (((1,), (0,)), ((), ())),
        preferred_element_type=jnp.float32)
    d2 = r2_ref[...] - cross2 + w2_ref[...]
    take = jnp.logical_or(mj == 0, d2 < bd_ref[...])
    bi_ref[...] = jnp.where(take, jnp.int32(mj * _TM), bi_ref[...])
    bd_ref[...] = jnp.where(take, d2, bd_ref[...])

    @pl.when(mj == pl.num_programs(0) - 1)
    def _emit():
        bd = bd_ref[...]
        dmin = jnp.min(bd, axis=1, keepdims=True)
        jj = lax.broadcasted_iota(jnp.int32, bd.shape, 1)
        gi = bi_ref[...] + jj
        ei = jnp.where(bd == dmin, gi, jnp.int32(_M))
        idx_ref[...] = jnp.min(ei, axis=1, keepdims=True)


def _argmin_call(res, wt2, r2, w2):
    n = res.shape[0]
    out = pl.pallas_call(
        _argmin_body,
        grid=(_NM,),
        in_specs=[
            pl.BlockSpec((n, _D), lambda j: (0, 0)),
            pl.BlockSpec((_D, _TM), lambda j: (0, j)),
            pl.BlockSpec((n, 1), lambda j: (0, 0)),
            pl.BlockSpec((1, _TM), lambda j: (0, j)),
        ],
        out_specs=pl.BlockSpec((n, 1), lambda j: (0, 0)),
        out_shape=jax.ShapeDtypeStruct((n, 1), jnp.int32),
        scratch_shapes=[
            pltpu.VMEM((n, _TM), jnp.float32),
            pltpu.VMEM((n, _TM), jnp.int32),
        ],
        compiler_params=pltpu.CompilerParams(
            dimension_semantics=("arbitrary",)),
    )(res, wt2, r2, w2)
    return out.reshape(n)


_NC, _NS, _L = 2, 16, 16
_NW = _NC * _NS

_sc_mesh = plsc.VectorSubcoreMesh(core_axis_name="c", subcore_axis_name="s")


def _pick_chunk(pw):
    for ch in (144, 96, 72, 48, 24, 8):
        if pw % ch == 0:
            return ch
    return pw


def _scsub_body(pw, ch, w_hbm, idx_hbm, res_hbm, out_hbm, idxv, rowsv, resv,
                sem):
    wid = lax.axis_index("s") * _NC + lax.axis_index("c")
    base = wid * pw
    for c in range(pw // ch):
        off = base + c * ch
        pltpu.sync_copy(idx_hbm.at[pl.ds(off, ch)], idxv)
        gather = pltpu.async_copy(w_hbm.at[idxv], rowsv, sem)
        pltpu.sync_copy(res_hbm.at[pl.ds(off, ch)], resv)
        gather.wait()

        def _row(r, carry):
            for j in range(_D // _L):
                sl = pl.ds(j * _L, _L)
                resv[r, sl] = resv[r, sl] - rowsv[r, sl]
            return carry

        lax.fori_loop(0, ch, _row, 0)
        pltpu.sync_copy(resv, out_hbm.at[pl.ds(off, ch)])


def _sc_residual_update(codebook, idx, res):
    n = res.shape[0]
    pw = n // _NW
    ch = _pick_chunk(pw)
    fn = pl.kernel(
        functools.partial(_scsub_body, pw, ch),
        out_type=jax.ShapeDtypeStruct((n, _D), jnp.float32),
        mesh=_sc_mesh,
        scratch_types=[
            pltpu.VMEM((ch,), jnp.int32),
            pltpu.VMEM((ch, _D), jnp.float32),
            pltpu.VMEM((ch, _D), jnp.float32),
            pltpu.SemaphoreType.DMA,
        ],
    )
    return fn(codebook, idx, res)


def kernel(x, codebooks):
    x2 = x.reshape(_N, _D)
    nk = codebooks.shape[0]
    wt2s = [(2.0 * codebooks[k]).T for k in range(nk)]
    w2s = [jnp.sum(codebooks[k] * codebooks[k], axis=-1).reshape(1, _M)
           for k in range(nk)]
    span = _N // _NCHAIN
    chain_idx = []
    chain_res = []
    for h in range(_NCHAIN):
        residual = x2[h * span:(h + 1) * span]
        idxs = []
        for k in range(nk):
            r2 = jnp.sum(residual * residual, axis=-1, keepdims=True)
            idx = _argmin_call(residual, wt2s[k], r2, w2s[k])
            idxs.append(idx)
            residual = _sc_residual_update(codebooks[k], idx, residual)
        chain_idx.append(idxs)
        chain_res.append(residual)
    quantized = (x2 - jnp.concatenate(chain_res, axis=0)).reshape(_B, _S, _D)
    indices = jnp.stack(
        [jnp.concatenate([chain_idx[h][k] for h in range(_NCHAIN)]).reshape(_B, _S)
         for k in range(nk)], axis=0)
    return (quantized, indices)


# speedup vs baseline: 1.5029x; 1.1109x over previous
import functools

import jax
import jax.numpy as jnp
from jax import lax
from jax.experimental import pallas as pl
from jax.experimental.pallas import tpu as pltpu
from jax.experimental.pallas import tpu_sc as plsc

_B, _S, _D = 16, 576, 256
_N = _B * _S
_M = 8192
_TM = 256
_NM = _M // _TM
_NCHAIN = 1


def _argmin_body(res_ref, wt2_ref, r2_ref, w2_ref, idx_ref, bd_ref, bi_ref):
    mj = pl.program_id(0)

    cross2 = lax.dot_general(
        res_ref[...], wt2_ref[...], (((1,), (1,)), ((), ())),
        preferred_element_type=jnp.float32)
    d2 = r2_ref[...] - cross2 + w2_ref[...]
    take = jnp.logical_or(mj == 0, d2 < bd_ref[...])
    bi_ref[...] = jnp.where(take, jnp.int32(mj * _TM), bi_ref[...])
    bd_ref[...] = jnp.where(take, d2, bd_ref[...])

    @pl.when(mj == pl.num_programs(0) - 1)
    def _emit():
        bd = bd_ref[...]
        dmin = jnp.min(bd, axis=1, keepdims=True)
        jj = lax.broadcasted_iota(jnp.int32, bd.shape, 1)
        gi = bi_ref[...] + jj
        ei = jnp.where(bd == dmin, gi, jnp.int32(_M))
        idx_ref[...] = jnp.min(ei, axis=1, keepdims=True)


def _argmin_call(res, wt2, r2, w2):
    n = res.shape[0]
    out = pl.pallas_call(
        _argmin_body,
        grid=(_NM,),
        in_specs=[
            pl.BlockSpec((n, _D), lambda j: (0, 0)),
            pl.BlockSpec((_TM, _D), lambda j: (j, 0)),
            pl.BlockSpec((n, 1), lambda j: (0, 0)),
            pl.BlockSpec((1, _TM), lambda j: (0, j)),
        ],
        out_specs=pl.BlockSpec((n, 1), lambda j: (0, 0)),
        out_shape=jax.ShapeDtypeStruct((n, 1), jnp.int32),
        scratch_shapes=[
            pltpu.VMEM((n, _TM), jnp.float32),
            pltpu.VMEM((n, _TM), jnp.int32),
        ],
        compiler_params=pltpu.CompilerParams(
            dimension_semantics=("arbitrary",)),
    )(res, wt2, r2, w2)
    return out.reshape(n)


_NC, _NS, _L = 2, 16, 16
_NW = _NC * _NS

_sc_mesh = plsc.VectorSubcoreMesh(core_axis_name="c", subcore_axis_name="s")


def _pick_chunk(pw):
    for ch in (144, 96, 72, 48, 24, 8):
        if pw % ch == 0:
            return ch
    return pw


def _scsub_body(pw, ch, w_hbm, idx_hbm, res_hbm, out_hbm, idxv, rowsv, resv,
                sem):
    wid = lax.axis_index("s") * _NC + lax.axis_index("c")
    base = wid * pw
    for c in range(pw // ch):
        off = base + c * ch
        pltpu.sync_copy(idx_hbm.at[pl.ds(off, ch)], idxv)
        gather = pltpu.async_copy(w_hbm.at[idxv], rowsv, sem)
        pltpu.sync_copy(res_hbm.at[pl.ds(off, ch)], resv)
        gather.wait()

        def _row(r, carry):
            for j in range(_D // _L):
                sl = pl.ds(j * _L, _L)
                resv[r, sl] = resv[r, sl] - rowsv[r, sl]
            return carry

        lax.fori_loop(0, ch, _row, 0)
        pltpu.sync_copy(resv, out_hbm.at[pl.ds(off, ch)])


def _sc_residual_update(codebook, idx, res):
    n = res.shape[0]
    pw = n // _NW
    ch = _pick_chunk(pw)
    fn = pl.kernel(
        functools.partial(_scsub_body, pw, ch),
        out_type=jax.ShapeDtypeStruct((n, _D), jnp.float32),
        mesh=_sc_mesh,
        scratch_types=[
            pltpu.VMEM((ch,), jnp.int32),
            pltpu.VMEM((ch, _D), jnp.float32),
            pltpu.VMEM((ch, _D), jnp.float32),
            pltpu.SemaphoreType.DMA,
        ],
    )
    return fn(codebook, idx, res)


def kernel(x, codebooks):
    x2 = x.reshape(_N, _D)
    nk = codebooks.shape[0]
    wt2s = [2.0 * codebooks[k] for k in range(nk)]
    w2s = [jnp.sum(codebooks[k] * codebooks[k], axis=-1).reshape(1, _M)
           for k in range(nk)]
    span = _N // _NCHAIN
    chain_idx = []
    chain_res = []
    for h in range(_NCHAIN):
        residual = x2[h * span:(h + 1) * span]
        idxs = []
        for k in range(nk):
            r2 = jnp.sum(residual * residual, axis=-1, keepdims=True)
            idx = _argmin_call(residual, wt2s[k], r2, w2s[k])
            idxs.append(idx)
            residual = _sc_residual_update(codebooks[k], idx, residual)
        chain_idx.append(idxs)
        chain_res.append(residual)
    quantized = (x2 - jnp.concatenate(chain_res, axis=0)).reshape(_B, _S, _D)
    indices = jnp.stack(
        [jnp.concatenate([chain_idx[h][k] for h in range(_NCHAIN)]).reshape(_B, _S)
         for k in range(nk)], axis=0)
    return (quantized, indices)
